# Initial kernel scaffold; baseline (speedup 1.0000x reference)
import jax
import jax.numpy as jnp
from jax.experimental import pallas as pl


def kernel(x, edge_index, edge_attr, batch, W_node, b_node, W_edge, b_edge, W_g1, b_g1, W_g2, b_g2, W_g3, b_g3, W_c1, b_c1, W_c2, b_c2):
    raise NotImplementedError("write your pallas kernel here")


# speedup vs baseline: 7.6249x; 7.6249x over previous
import jax
import jax.numpy as jnp
from jax import lax
from jax.experimental import pallas as pl
from jax.experimental.pallas import tpu as pltpu
from jax.experimental.pallas import tpu_sc as plsc

N = 10000
E = 320000
D = 128
H = 64
B = 64
DE = 16

NC = 2
NS = 16
NW = NC * NS
CH = 128
KPW = 79
EPW = KPW * CH
EPAD = EPW * NW
KPT = 158
RPT = 626
NSP = RPT * NS
NH = 5000
NHP = 5008
RHT = 313


def _sc_edge_stats_body(src_hbm, dst_hbm, ea_hbm, od_hbm, os_hbm, z16_hbm,
                        attr_out, degcnt_out,
                        sidx, didx, sloc, dloc, eabuf, onesd, oness, stage,
                        attr_sh, degcnt_sh):
    c = lax.axis_index("c")
    s = lax.axis_index("s")
    r0 = s * RHT
    nbase = c * NH

    pltpu.sync_copy(od_hbm, onesd)
    pltpu.sync_copy(os_hbm, oness)

    pltpu.sync_copy(z16_hbm.at[pl.ds(r0, RHT)], stage)
    pltpu.sync_copy(stage, attr_sh.at[pl.ds(r0, RHT)])
    pltpu.sync_copy(stage, degcnt_sh.at[pl.ds(r0, RHT)])
    plsc.subcore_barrier()

    def localize(gidx, lidx):
        for j in range(CH // 16):
            g = gidx[pl.ds(j * 16, 16)]
            l = g - nbase
            ok = (l >= 0) & (l < NH)
            lidx[pl.ds(j * 16, 16)] = jnp.where(ok, l, NH)

    def step(k, _):
        off = s * (KPT * CH) + k * CH
        pltpu.sync_copy(src_hbm.at[pl.ds(off, CH)], sidx)
        pltpu.sync_copy(dst_hbm.at[pl.ds(off, CH)], didx)
        pltpu.sync_copy(ea_hbm.at[pl.ds(off, CH)], eabuf)
        localize(sidx, sloc)
        localize(didx, dloc)
        pltpu.sync_copy(eabuf, attr_sh.at[sloc], add=True)
        pltpu.sync_copy(onesd, degcnt_sh.at[dloc], add=True)
        pltpu.sync_copy(oness, degcnt_sh.at[sloc], add=True)
        return ()

    lax.fori_loop(0, KPT, step, (), unroll=False)
    plsc.subcore_barrier()

    pltpu.sync_copy(attr_sh.at[pl.ds(r0, RHT)], stage)
    pltpu.sync_copy(stage, attr_out.at[c, pl.ds(r0, RHT)])
    pltpu.sync_copy(degcnt_sh.at[pl.ds(r0, RHT)], stage)
    pltpu.sync_copy(stage, degcnt_out.at[c, pl.ds(r0, RHT)])


_sc_edge_stats = pl.kernel(
    _sc_edge_stats_body,
    out_type=(
        jax.ShapeDtypeStruct((NC, NHP, DE), jnp.float32),
        jax.ShapeDtypeStruct((NC, NHP, DE), jnp.float32),
    ),
    mesh=plsc.VectorSubcoreMesh(core_axis_name="c", subcore_axis_name="s"),
    scratch_types=(
        pltpu.VMEM((CH,), jnp.int32),
        pltpu.VMEM((CH,), jnp.int32),
        pltpu.VMEM((CH,), jnp.int32),
        pltpu.VMEM((CH,), jnp.int32),
        pltpu.VMEM((CH, DE), jnp.float32),
        pltpu.VMEM((CH, DE), jnp.float32),
        pltpu.VMEM((CH, DE), jnp.float32),
        pltpu.VMEM((RHT, DE), jnp.float32),
        pltpu.VMEM_SHARED((NHP, DE), jnp.float32),
        pltpu.VMEM_SHARED((NHP, DE), jnp.float32),
    ),
    compiler_params=pltpu.CompilerParams(use_tc_tiling_on_sc=False),
)


def _sc_aggregate_body(src_hbm, dst_hbm, mp_hbm, z64_hbm,
                       acc_out,
                       sidx, didx, rows, stage, sem,
                       acc_sh):
    c = lax.axis_index("c")
    s = lax.axis_index("s")
    wid = s * NC + c
    base = wid * EPW
    r0 = s * RPT

    pltpu.sync_copy(z64_hbm.at[pl.ds(r0, RPT)], stage)
    pltpu.sync_copy(stage, acc_sh.at[pl.ds(r0, RPT)])
    plsc.subcore_barrier()

    def step(k, _):
        off = base + k * CH
        pltpu.sync_copy(src_hbm.at[pl.ds(off, CH)], sidx)
        pltpu.sync_copy(dst_hbm.at[pl.ds(off, CH)], didx)
        pltpu.async_copy(mp_hbm.at[sidx], rows, sem).wait()
        pltpu.sync_copy(rows, acc_sh.at[didx], add=True)
        return ()

    lax.fori_loop(0, KPW, step, (), unroll=False)
    plsc.subcore_barrier()

    pltpu.sync_copy(acc_sh.at[pl.ds(r0, RPT)], stage)
    pltpu.sync_copy(stage, acc_out.at[c, pl.ds(r0, RPT)])


_sc_aggregate = pl.kernel(
    _sc_aggregate_body,
    out_type=jax.ShapeDtypeStruct((NC, NSP, H), jnp.float32),
    mesh=plsc.VectorSubcoreMesh(core_axis_name="c", subcore_axis_name="s"),
    scratch_types=(
        pltpu.VMEM((CH,), jnp.int32),
        pltpu.VMEM((CH,), jnp.int32),
        pltpu.VMEM((CH, H), jnp.float32),
        pltpu.VMEM((RPT, H), jnp.float32),
        pltpu.SemaphoreType.DMA,
        pltpu.VMEM_SHARED((NSP, H), jnp.float32),
    ),
    compiler_params=pltpu.CompilerParams(use_tc_tiling_on_sc=False),
)


_PREC = lax.Precision.HIGHEST


def _dot(a, b):
    return jnp.dot(a, b, precision=_PREC, preferred_element_type=jnp.float32)


def _tc_dinv_body(degcnt_ref, seld_ref, sels_ref, dinv_ref, cnt_ref):
    dc = jnp.concatenate(
        [degcnt_ref[0, :NH, :], degcnt_ref[1, :NH, :]], axis=0)
    deg = _dot(dc, seld_ref[...]) + 1.0
    dinv_ref[:N] = lax.rsqrt(deg)
    dinv_ref[N:] = jnp.zeros((NSP - N, 1), jnp.float32)
    cnt_ref[:N] = _dot(dc, sels_ref[...])
    cnt_ref[N:] = jnp.zeros((NSP - N, 1), jnp.float32)


_tc_dinv = pl.pallas_call(
    _tc_dinv_body,
    out_shape=(
        jax.ShapeDtypeStruct((NSP, 1), jnp.float32),
        jax.ShapeDtypeStruct((NSP, 1), jnp.float32),
    ),
)


def _tc_embed_body(x_ref, attr_ref, dinv_ref, cnt_ref, wn_ref, bn_ref,
                   wea_ref, be_ref, wg1_ref, mp1_ref):
    a16 = jnp.concatenate(
        [attr_ref[0, :NH, :], attr_ref[1, :NH, :]], axis=0)
    h = (_dot(x_ref[...], wn_ref[...]) + bn_ref[...]
         + _dot(a16, wea_ref[...]) + cnt_ref[:N] * be_ref[...])
    mp1_ref[:N] = dinv_ref[:N] * _dot(h, wg1_ref[...])
    mp1_ref[N:] = jnp.zeros((NSP - N, H), jnp.float32)


_tc_embed = pl.pallas_call(
    _tc_embed_body,
    out_shape=jax.ShapeDtypeStruct((NSP, H), jnp.float32),
)


def _tc_mid_body(acc_ref, mp_ref, dinv_ref, b_ref, w_ref, out_ref):
    a = acc_ref[0, :N, :] + acc_ref[1, :N, :] + mp_ref[:N, :]
    dinv = dinv_ref[:N, :]
    o = jnp.maximum(dinv * a + b_ref[...], 0.0)
    out_ref[:N] = dinv * _dot(o, w_ref[...])
    out_ref[N:] = jnp.zeros((NSP - N, H), jnp.float32)


_tc_mid = pl.pallas_call(
    _tc_mid_body,
    out_shape=jax.ShapeDtypeStruct((NSP, H), jnp.float32),
)


def _tc_head_body(acc_ref, mp_ref, dinv_ref, b3_ref, bat_ref,
                  wc1_ref, bc1_ref, wc2_ref, bc2_ref, out_ref):
    a = acc_ref[0, :N, :] + acc_ref[1, :N, :] + mp_ref[:N, :]
    h3 = dinv_ref[:N, :] * a + b3_ref[...]
    gid = lax.broadcasted_iota(jnp.int32, (B, N), 0)
    oh = (bat_ref[...] == gid).astype(jnp.float32)
    sums = _dot(oh, h3)
    cnts = jnp.sum(oh, axis=1, keepdims=True)
    pooled = sums / jnp.maximum(cnts, 1.0)
    z = jnp.maximum(_dot(pooled, wc1_ref[...]) + bc1_ref[...], 0.0)
    out_ref[...] = _dot(z, wc2_ref[...]) + bc2_ref[...]


_tc_head = pl.pallas_call(
    _tc_head_body,
    out_shape=jax.ShapeDtypeStruct((B, 1), jnp.float32),
)


def kernel(x, edge_index, edge_attr, batch,
           W_node, b_node, W_edge, b_edge,
           W_g1, b_g1, W_g2, b_g2, W_g3, b_g3,
           W_c1, b_c1, W_c2, b_c2):
    src = edge_index[0]
    dst = edge_index[1]
    padi = jnp.full((EPAD - E,), N, jnp.int32)
    srcp = jnp.concatenate([src, padi])
    dstp = jnp.concatenate([dst, padi])
    eap = jnp.concatenate(
        [edge_attr, jnp.zeros((EPAD - E, DE), jnp.float32)], axis=0)

    onesd = jnp.concatenate(
        [jnp.ones((CH, 8), jnp.float32), jnp.zeros((CH, 8), jnp.float32)],
        axis=1)
    oness = jnp.concatenate(
        [jnp.zeros((CH, 8), jnp.float32), jnp.ones((CH, 8), jnp.float32)],
        axis=1)
    z16 = jnp.zeros((NHP, DE), jnp.float32)
    z64 = jnp.zeros((NSP, H), jnp.float32)
    seld = jnp.zeros((DE, 1), jnp.float32).at[0, 0].set(1.0)
    sels = jnp.zeros((DE, 1), jnp.float32).at[8, 0].set(1.0)

    attr2, degcnt2 = _sc_edge_stats(srcp, dstp, eap, onesd, oness, z16)
    dinv, cnt = _tc_dinv(degcnt2, seld, sels)
    mp1 = _tc_embed(x, attr2, dinv, cnt, W_node, b_node[None, :],
                    W_edge, b_edge[None, :], W_g1)

    acc1 = _sc_aggregate(srcp, dstp, mp1, z64)
    mp2 = _tc_mid(acc1, mp1, dinv, b_g1[None, :], W_g2)
    acc2 = _sc_aggregate(srcp, dstp, mp2, z64)
    mp3 = _tc_mid(acc2, mp2, dinv, b_g2[None, :], W_g3)
    acc3 = _sc_aggregate(srcp, dstp, mp3, z64)

    out = _tc_head(acc3, mp3, dinv, b_g3[None, :], batch[None, :],
                   W_c1, b_c1[None, :], W_c2, b_c2[None, :])
    return out[:, 0]


# speedup vs baseline: 7.8588x; 1.0307x over previous
import jax
import jax.numpy as jnp
from jax import lax
from jax.experimental import pallas as pl
from jax.experimental.pallas import tpu as pltpu
from jax.experimental.pallas import tpu_sc as plsc

N = 10000
E = 320000
D = 128
H = 64
B = 64
DE = 16

NC = 2
NS = 16
NW = NC * NS
CH = 128
KPW = 80
EPW = KPW * CH
EPAD = EPW * NW
KPT = 160
NBUF = 8
PF = 4
RND = KPW // NBUF
RPT = 626
NSP = RPT * NS
NH = 5000
NHP = 5008
RHT = 313


def _sc_edge_stats_body(src_hbm, dst_hbm, ea_hbm, od_hbm, os_hbm, z16_hbm,
                        attr_out, degcnt_out,
                        sidx, didx, sloc, dloc, eabuf, onesd, oness, stage,
                        attr_sh, degcnt_sh):
    c = lax.axis_index("c")
    s = lax.axis_index("s")
    r0 = s * RHT
    nbase = c * NH

    pltpu.sync_copy(od_hbm, onesd)
    pltpu.sync_copy(os_hbm, oness)

    pltpu.sync_copy(z16_hbm.at[pl.ds(r0, RHT)], stage)
    pltpu.sync_copy(stage, attr_sh.at[pl.ds(r0, RHT)])
    pltpu.sync_copy(stage, degcnt_sh.at[pl.ds(r0, RHT)])
    plsc.subcore_barrier()

    def localize(gidx, lidx):
        for j in range(CH // 16):
            g = gidx[0, pl.ds(j * 16, 16)]
            l = g - nbase
            ok = (l >= 0) & (l < NH)
            lidx[pl.ds(j * 16, 16)] = jnp.where(ok, l, NH)

    def step(k, _):
        row = s * KPT + k
        pltpu.sync_copy(src_hbm.at[pl.ds(row, 1)], sidx)
        pltpu.sync_copy(dst_hbm.at[pl.ds(row, 1)], didx)
        pltpu.sync_copy(ea_hbm.at[pl.ds(row * CH, CH)], eabuf)
        localize(sidx, sloc)
        localize(didx, dloc)
        pltpu.sync_copy(eabuf, attr_sh.at[sloc], add=True)
        pltpu.sync_copy(onesd, degcnt_sh.at[dloc], add=True)
        pltpu.sync_copy(oness, degcnt_sh.at[sloc], add=True)
        return ()

    lax.fori_loop(0, KPT, step, (), unroll=False)
    plsc.subcore_barrier()

    pltpu.sync_copy(attr_sh.at[pl.ds(r0, RHT)], stage)
    pltpu.sync_copy(stage, attr_out.at[c, pl.ds(r0, RHT)])
    pltpu.sync_copy(degcnt_sh.at[pl.ds(r0, RHT)], stage)
    pltpu.sync_copy(stage, degcnt_out.at[c, pl.ds(r0, RHT)])


_sc_edge_stats = pl.kernel(
    _sc_edge_stats_body,
    out_type=(
        jax.ShapeDtypeStruct((NC, NHP, DE), jnp.float32),
        jax.ShapeDtypeStruct((NC, NHP, DE), jnp.float32),
    ),
    mesh=plsc.VectorSubcoreMesh(core_axis_name="c", subcore_axis_name="s"),
    scratch_types=(
        pltpu.VMEM((1, CH), jnp.int32),
        pltpu.VMEM((1, CH), jnp.int32),
        pltpu.VMEM((CH,), jnp.int32),
        pltpu.VMEM((CH,), jnp.int32),
        pltpu.VMEM((CH, DE), jnp.float32),
        pltpu.VMEM((CH, DE), jnp.float32),
        pltpu.VMEM((CH, DE), jnp.float32),
        pltpu.VMEM((RHT, DE), jnp.float32),
        pltpu.VMEM_SHARED((NHP, DE), jnp.float32),
        pltpu.VMEM_SHARED((NHP, DE), jnp.float32),
    ),
    compiler_params=pltpu.CompilerParams(use_tc_tiling_on_sc=False),
)


def _sc_aggregate_body(src_hbm, dst_hbm, mp_hbm, z64_hbm,
                       acc_out,
                       sidx, didx, r0b, r1b, r2b, r3b, r4b, r5b, r6b, r7b,
                       gsem, ssem,
                       acc_sh):
    c = lax.axis_index("c")
    s = lax.axis_index("s")
    wid = s * NC + c
    kbase = wid * KPW
    r0 = s * RPT
    rows = (r0b, r1b, r2b, r3b, r4b, r5b, r6b, r7b)

    pltpu.sync_copy(src_hbm.at[pl.ds(kbase, KPW)], sidx)
    pltpu.sync_copy(dst_hbm.at[pl.ds(kbase, KPW)], didx)

    for t, sz in ((0, CH), (1, CH), (2, CH), (3, CH), (4, RPT - 4 * CH)):
        pltpu.sync_copy(z64_hbm.at[pl.ds(r0 + t * CH, sz)],
                        rows[0].at[pl.ds(0, sz)])
        pltpu.sync_copy(rows[0].at[pl.ds(0, sz)],
                        acc_sh.at[pl.ds(r0 + t * CH, sz)])
    plsc.subcore_barrier()

    def wait_gather(j):
        pltpu.make_async_copy(mp_hbm.at[sidx.at[j]], rows[j], gsem.at[j]).wait()

    def wait_scatter(b):
        pltpu.make_async_copy(rows[b], acc_sh.at[didx.at[b]], ssem.at[b]).wait()

    def start_gather(b, kk):
        pltpu.async_copy(mp_hbm.at[sidx.at[kk]], rows[b], gsem.at[b])

    def start_scatter(j, k):
        pltpu.async_copy(rows[j], acc_sh.at[didx.at[k]], ssem.at[j], add=True)

    for j in range(PF):
        start_gather(j, j)

    for j in range(NBUF):
        wait_gather(j)
        start_scatter(j, j)
        b = (j + PF) % NBUF
        if j < NBUF - PF:
            start_gather(b, j + PF)
        else:
            wait_scatter(b)
            start_gather(b, j + PF)

    def rbody(r, _):
        for j in range(NBUF):
            k = r * NBUF + j
            wait_gather(j)
            start_scatter(j, k)
            b = (j + PF) % NBUF
            wait_scatter(b)
            start_gather(b, k + PF)
        return ()

    lax.fori_loop(1, RND - 1, rbody, (), unroll=False)

    for j in range(NBUF):
        k = (RND - 1) * NBUF + j
        wait_gather(j)
        start_scatter(j, k)
        if j < PF:
            b = (j + PF) % NBUF
            wait_scatter(b)
            start_gather(b, k + PF)

    for j in range(NBUF):
        wait_scatter(j)
    plsc.subcore_barrier()

    for t, sz in ((0, CH), (1, CH), (2, CH), (3, CH), (4, RPT - 4 * CH)):
        pltpu.sync_copy(acc_sh.at[pl.ds(r0 + t * CH, sz)],
                        rows[1].at[pl.ds(0, sz)])
        pltpu.sync_copy(rows[1].at[pl.ds(0, sz)],
                        acc_out.at[c, pl.ds(r0 + t * CH, sz)])


_sc_aggregate = pl.kernel(
    _sc_aggregate_body,
    out_type=jax.ShapeDtypeStruct((NC, NSP, H), jnp.float32),
    mesh=plsc.VectorSubcoreMesh(core_axis_name="c", subcore_axis_name="s"),
    scratch_types=(
        pltpu.VMEM((KPW, CH), jnp.int32),
        pltpu.VMEM((KPW, CH), jnp.int32),
    ) + tuple(pltpu.VMEM((CH, H), jnp.float32) for _ in range(NBUF)) + (
        pltpu.SemaphoreType.DMA((NBUF,)),
        pltpu.SemaphoreType.DMA((NBUF,)),
        pltpu.VMEM_SHARED((NSP, H), jnp.float32),
    ),
    compiler_params=pltpu.CompilerParams(use_tc_tiling_on_sc=False),
)


_PREC = lax.Precision.HIGHEST


def _dot(a, b):
    return jnp.dot(a, b, precision=_PREC, preferred_element_type=jnp.float32)


def _r(t):
    return t.astype(jnp.bfloat16).astype(jnp.float32)


def _dotd(a, b):
    return _dot(_r(a), _r(b))


def _tc_dinv_body(degcnt_ref, seld_ref, sels_ref, dinv_ref, cnt_ref):
    dc = jnp.concatenate(
        [degcnt_ref[0, :NH, :], degcnt_ref[1, :NH, :]], axis=0)
    deg = _dot(dc, seld_ref[...]) + 1.0
    dinv_ref[:N] = 1.0 / jnp.sqrt(deg)
    dinv_ref[N:] = jnp.zeros((NSP - N, 1), jnp.float32)
    cnt_ref[:N] = _dot(dc, sels_ref[...])
    cnt_ref[N:] = jnp.zeros((NSP - N, 1), jnp.float32)


_tc_dinv = pl.pallas_call(
    _tc_dinv_body,
    out_shape=(
        jax.ShapeDtypeStruct((NSP, 1), jnp.float32),
        jax.ShapeDtypeStruct((NSP, 1), jnp.float32),
    ),
)


def _tc_embed_body(x_ref, attr_ref, dinv_ref, cnt_ref, wn_ref, bn_ref,
                   wea_ref, be_ref, wg1_ref, mp1_ref):
    a16 = jnp.concatenate(
        [attr_ref[0, :NH, :], attr_ref[1, :NH, :]], axis=0)
    h = (_dotd(x_ref[...], wn_ref[...]) + bn_ref[...]
         + _dot(a16, _r(wea_ref[...])) + cnt_ref[:N] * be_ref[...])
    mp1_ref[:N] = dinv_ref[:N] * _dotd(h, wg1_ref[...])
    mp1_ref[N:] = jnp.zeros((NSP - N, H), jnp.float32)


_tc_embed = pl.pallas_call(
    _tc_embed_body,
    out_shape=jax.ShapeDtypeStruct((NSP, H), jnp.float32),
)


def _tc_mid_body(acc_ref, mp_ref, dinv_ref, b_ref, w_ref, out_ref):
    a = acc_ref[0, :N, :] + acc_ref[1, :N, :] + mp_ref[:N, :]
    dinv = dinv_ref[:N, :]
    o = jnp.maximum(dinv * a + b_ref[...], 0.0)
    out_ref[:N] = dinv * _dotd(o, w_ref[...])
    out_ref[N:] = jnp.zeros((NSP - N, H), jnp.float32)


_tc_mid = pl.pallas_call(
    _tc_mid_body,
    out_shape=jax.ShapeDtypeStruct((NSP, H), jnp.float32),
)


def _tc_head_body(acc_ref, mp_ref, dinv_ref, b3_ref, bat_ref,
                  wc1_ref, bc1_ref, wc2_ref, bc2_ref, out_ref):
    a = acc_ref[0, :N, :] + acc_ref[1, :N, :] + mp_ref[:N, :]
    h3 = dinv_ref[:N, :] * a + b3_ref[...]
    gid = lax.broadcasted_iota(jnp.int32, (B, N), 0)
    oh = (bat_ref[...] == gid).astype(jnp.float32)
    sums = _dot(oh, h3)
    cnts = jnp.sum(oh, axis=1, keepdims=True)
    pooled = sums / jnp.maximum(cnts, 1.0)
    z = jnp.maximum(_dotd(pooled, wc1_ref[...]) + bc1_ref[...], 0.0)
    out_ref[...] = _dotd(z, wc2_ref[...]) + bc2_ref[...]


_tc_head = pl.pallas_call(
    _tc_head_body,
    out_shape=jax.ShapeDtypeStruct((B, 1), jnp.float32),
)


def kernel(x, edge_index, edge_attr, batch,
           W_node, b_node, W_edge, b_edge,
           W_g1, b_g1, W_g2, b_g2, W_g3, b_g3,
           W_c1, b_c1, W_c2, b_c2):
    src = edge_index[0]
    dst = edge_index[1]
    padi = jnp.full((EPAD - E,), N, jnp.int32)
    srcp = jnp.concatenate([src, padi]).reshape(NW * KPW, CH)
    dstp = jnp.concatenate([dst, padi]).reshape(NW * KPW, CH)
    eap = jnp.concatenate(
        [edge_attr.astype(jnp.bfloat16).astype(jnp.float32),
         jnp.zeros((EPAD - E, DE), jnp.float32)], axis=0)

    onesd = jnp.concatenate(
        [jnp.ones((CH, 8), jnp.float32), jnp.zeros((CH, 8), jnp.float32)],
        axis=1)
    oness = jnp.concatenate(
        [jnp.zeros((CH, 8), jnp.float32), jnp.ones((CH, 8), jnp.float32)],
        axis=1)
    z16 = jnp.zeros((NHP, DE), jnp.float32)
    z64 = jnp.zeros((NSP, H), jnp.float32)
    seld = jnp.zeros((DE, 1), jnp.float32).at[0, 0].set(1.0)
    sels = jnp.zeros((DE, 1), jnp.float32).at[8, 0].set(1.0)

    attr2, degcnt2 = _sc_edge_stats(srcp, dstp, eap, onesd, oness, z16)
    dinv, cnt = _tc_dinv(degcnt2, seld, sels)
    mp1 = _tc_embed(x, attr2, dinv, cnt, W_node, b_node[None, :],
                    W_edge, b_edge[None, :], W_g1)

    acc1 = _sc_aggregate(srcp, dstp, mp1, z64)
    mp2 = _tc_mid(acc1, mp1, dinv, b_g1[None, :], W_g2)
    acc2 = _sc_aggregate(srcp, dstp, mp2, z64)
    mp3 = _tc_mid(acc2, mp2, dinv, b_g2[None, :], W_g3)
    acc3 = _sc_aggregate(srcp, dstp, mp3, z64)

    out = _tc_head(acc3, mp3, dinv, b_g3[None, :], batch[None, :],
                   W_c1, b_c1[None, :], W_c2, b_c2[None, :])
    return out[:, 0]


# speedup vs baseline: 7.8654x; 1.0008x over previous
import jax
import jax.numpy as jnp
from jax import lax
from jax.experimental import pallas as pl
from jax.experimental.pallas import tpu as pltpu
from jax.experimental.pallas import tpu_sc as plsc

N = 10000
E = 320000
D = 128
H = 64
B = 64
DE = 16

NC = 2
NS = 16
NW = NC * NS
CH = 128
KPW = 80
EPW = KPW * CH
EPAD = EPW * NW
KPT = 160
NBUF = 8
PF = 6
RND = KPW // NBUF
RPT = 626
NSP = RPT * NS
NH = 5000
NHP = 5008
RHT = 313


def _sc_edge_stats_body(src_hbm, dst_hbm, ea_hbm, od_hbm, os_hbm, z16_hbm,
                        attr_out, degcnt_out,
                        sidx, didx, sloc, dloc, eabuf, onesd, oness, stage,
                        attr_sh, degcnt_sh):
    c = lax.axis_index("c")
    s = lax.axis_index("s")
    r0 = s * RHT
    nbase = c * NH

    pltpu.sync_copy(od_hbm, onesd)
    pltpu.sync_copy(os_hbm, oness)

    pltpu.sync_copy(z16_hbm.at[pl.ds(r0, RHT)], stage)
    pltpu.sync_copy(stage, attr_sh.at[pl.ds(r0, RHT)])
    pltpu.sync_copy(stage, degcnt_sh.at[pl.ds(r0, RHT)])
    plsc.subcore_barrier()

    def localize(gidx, lidx):
        for j in range(CH // 16):
            g = gidx[0, pl.ds(j * 16, 16)]
            l = g - nbase
            ok = (l >= 0) & (l < NH)
            lidx[pl.ds(j * 16, 16)] = jnp.where(ok, l, NH)

    def step(k, _):
        row = s * KPT + k
        pltpu.sync_copy(src_hbm.at[pl.ds(row, 1)], sidx)
        pltpu.sync_copy(dst_hbm.at[pl.ds(row, 1)], didx)
        pltpu.sync_copy(ea_hbm.at[pl.ds(row * CH, CH)], eabuf)
        localize(sidx, sloc)
        localize(didx, dloc)
        pltpu.sync_copy(eabuf, attr_sh.at[sloc], add=True)
        pltpu.sync_copy(onesd, degcnt_sh.at[dloc], add=True)
        pltpu.sync_copy(oness, degcnt_sh.at[sloc], add=True)
        return ()

    lax.fori_loop(0, KPT, step, (), unroll=False)
    plsc.subcore_barrier()

    pltpu.sync_copy(attr_sh.at[pl.ds(r0, RHT)], stage)
    pltpu.sync_copy(stage, attr_out.at[c, pl.ds(r0, RHT)])
    pltpu.sync_copy(degcnt_sh.at[pl.ds(r0, RHT)], stage)
    pltpu.sync_copy(stage, degcnt_out.at[c, pl.ds(r0, RHT)])


_sc_edge_stats = pl.kernel(
    _sc_edge_stats_body,
    out_type=(
        jax.ShapeDtypeStruct((NC, NHP, DE), jnp.float32),
        jax.ShapeDtypeStruct((NC, NHP, DE), jnp.float32),
    ),
    mesh=plsc.VectorSubcoreMesh(core_axis_name="c", subcore_axis_name="s"),
    scratch_types=(
        pltpu.VMEM((1, CH), jnp.int32),
        pltpu.VMEM((1, CH), jnp.int32),
        pltpu.VMEM((CH,), jnp.int32),
        pltpu.VMEM((CH,), jnp.int32),
        pltpu.VMEM((CH, DE), jnp.float32),
        pltpu.VMEM((CH, DE), jnp.float32),
        pltpu.VMEM((CH, DE), jnp.float32),
        pltpu.VMEM((RHT, DE), jnp.float32),
        pltpu.VMEM_SHARED((NHP, DE), jnp.float32),
        pltpu.VMEM_SHARED((NHP, DE), jnp.float32),
    ),
    compiler_params=pltpu.CompilerParams(use_tc_tiling_on_sc=False),
)


def _sc_aggregate_body(src_hbm, dst_hbm, mp_hbm, z64_hbm,
                       acc_out,
                       sidx, didx, r0b, r1b, r2b, r3b, r4b, r5b, r6b, r7b,
                       gsem, ssem,
                       acc_sh):
    c = lax.axis_index("c")
    s = lax.axis_index("s")
    wid = s * NC + c
    kbase = wid * KPW
    r0 = s * RPT
    rows = (r0b, r1b, r2b, r3b, r4b, r5b, r6b, r7b)

    pltpu.sync_copy(src_hbm.at[pl.ds(kbase, KPW)], sidx)
    pltpu.sync_copy(dst_hbm.at[pl.ds(kbase, KPW)], didx)

    for t, sz in ((0, CH), (1, CH), (2, CH), (3, CH), (4, RPT - 4 * CH)):
        pltpu.sync_copy(z64_hbm.at[pl.ds(r0 + t * CH, sz)],
                        rows[0].at[pl.ds(0, sz)])
        pltpu.sync_copy(rows[0].at[pl.ds(0, sz)],
                        acc_sh.at[pl.ds(r0 + t * CH, sz)])
    plsc.subcore_barrier()

    def wait_gather(j):
        pltpu.make_async_copy(mp_hbm.at[sidx.at[j]], rows[j], gsem.at[j]).wait()

    def wait_scatter(b):
        pltpu.make_async_copy(rows[b], acc_sh.at[didx.at[b]], ssem.at[b]).wait()

    def start_gather(b, kk):
        pltpu.async_copy(mp_hbm.at[sidx.at[kk]], rows[b], gsem.at[b])

    def start_scatter(j, k):
        pltpu.async_copy(rows[j], acc_sh.at[didx.at[k]], ssem.at[j], add=True)

    for j in range(PF):
        start_gather(j, j)

    for j in range(NBUF):
        wait_gather(j)
        start_scatter(j, j)
        b = (j + PF) % NBUF
        if j < NBUF - PF:
            start_gather(b, j + PF)
        else:
            wait_scatter(b)
            start_gather(b, j + PF)

    def rbody(r, _):
        for j in range(NBUF):
            k = r * NBUF + j
            wait_gather(j)
            start_scatter(j, k)
            b = (j + PF) % NBUF
            wait_scatter(b)
            start_gather(b, k + PF)
        return ()

    lax.fori_loop(1, RND - 1, rbody, (), unroll=False)

    for j in range(NBUF):
        k = (RND - 1) * NBUF + j
        wait_gather(j)
        start_scatter(j, k)
        if j < NBUF - PF:
            b = (j + PF) % NBUF
            wait_scatter(b)
            start_gather(b, k + PF)

    for j in range(NBUF):
        wait_scatter(j)
    plsc.subcore_barrier()

    for t, sz in ((0, CH), (1, CH), (2, CH), (3, CH), (4, RPT - 4 * CH)):
        pltpu.sync_copy(acc_sh.at[pl.ds(r0 + t * CH, sz)],
                        rows[1].at[pl.ds(0, sz)])
        pltpu.sync_copy(rows[1].at[pl.ds(0, sz)],
                        acc_out.at[c, pl.ds(r0 + t * CH, sz)])


_sc_aggregate = pl.kernel(
    _sc_aggregate_body,
    out_type=jax.ShapeDtypeStruct((NC, NSP, H), jnp.float32),
    mesh=plsc.VectorSubcoreMesh(core_axis_name="c", subcore_axis_name="s"),
    scratch_types=(
        pltpu.VMEM((KPW, CH), jnp.int32),
        pltpu.VMEM((KPW, CH), jnp.int32),
    ) + tuple(pltpu.VMEM((CH, H), jnp.float32) for _ in range(NBUF)) + (
        pltpu.SemaphoreType.DMA((NBUF,)),
        pltpu.SemaphoreType.DMA((NBUF,)),
        pltpu.VMEM_SHARED((NSP, H), jnp.float32),
    ),
    compiler_params=pltpu.CompilerParams(use_tc_tiling_on_sc=False),
)


_PREC = lax.Precision.HIGHEST


def _dot(a, b):
    return jnp.dot(a, b, precision=_PREC, preferred_element_type=jnp.float32)


def _r(t):
    return t.astype(jnp.bfloat16).astype(jnp.float32)


def _dotd(a, b):
    return _dot(_r(a), _r(b))


def _tc_dinv_body(degcnt_ref, seld_ref, sels_ref, dinv_ref, cnt_ref):
    dc = jnp.concatenate(
        [degcnt_ref[0, :NH, :], degcnt_ref[1, :NH, :]], axis=0)
    deg = _dot(dc, seld_ref[...]) + 1.0
    dinv_ref[:N] = 1.0 / jnp.sqrt(deg)
    dinv_ref[N:] = jnp.zeros((NSP - N, 1), jnp.float32)
    cnt_ref[:N] = _dot(dc, sels_ref[...])
    cnt_ref[N:] = jnp.zeros((NSP - N, 1), jnp.float32)


_tc_dinv = pl.pallas_call(
    _tc_dinv_body,
    out_shape=(
        jax.ShapeDtypeStruct((NSP, 1), jnp.float32),
        jax.ShapeDtypeStruct((NSP, 1), jnp.float32),
    ),
)


def _tc_embed_body(x_ref, attr_ref, dinv_ref, cnt_ref, wn_ref, bn_ref,
                   wea_ref, be_ref, wg1_ref, mp1_ref):
    a16 = jnp.concatenate(
        [attr_ref[0, :NH, :], attr_ref[1, :NH, :]], axis=0)
    h = (_dotd(x_ref[...], wn_ref[...]) + bn_ref[...]
         + _dot(a16, _r(wea_ref[...])) + cnt_ref[:N] * be_ref[...])
    mp1_ref[:N] = dinv_ref[:N] * _dotd(h, wg1_ref[...])
    mp1_ref[N:] = jnp.zeros((NSP - N, H), jnp.float32)


_tc_embed = pl.pallas_call(
    _tc_embed_body,
    out_shape=jax.ShapeDtypeStruct((NSP, H), jnp.float32),
)


def _tc_mid_body(acc_ref, mp_ref, dinv_ref, b_ref, w_ref, out_ref):
    a = acc_ref[0, :N, :] + acc_ref[1, :N, :] + mp_ref[:N, :]
    dinv = dinv_ref[:N, :]
    o = jnp.maximum(dinv * a + b_ref[...], 0.0)
    out_ref[:N] = dinv * _dotd(o, w_ref[...])
    out_ref[N:] = jnp.zeros((NSP - N, H), jnp.float32)


_tc_mid = pl.pallas_call(
    _tc_mid_body,
    out_shape=jax.ShapeDtypeStruct((NSP, H), jnp.float32),
)


def _tc_head_body(acc_ref, mp_ref, dinv_ref, b3_ref, bat_ref,
                  wc1_ref, bc1_ref, wc2_ref, bc2_ref, out_ref):
    a = acc_ref[0, :N, :] + acc_ref[1, :N, :] + mp_ref[:N, :]
    h3 = dinv_ref[:N, :] * a + b3_ref[...]
    gid = lax.broadcasted_iota(jnp.int32, (B, N), 0)
    oh = (bat_ref[...] == gid).astype(jnp.float32)
    sums = _dot(oh, h3)
    cnts = jnp.sum(oh, axis=1, keepdims=True)
    pooled = sums / jnp.maximum(cnts, 1.0)
    z = jnp.maximum(_dotd(pooled, wc1_ref[...]) + bc1_ref[...], 0.0)
    out_ref[...] = _dotd(z, wc2_ref[...]) + bc2_ref[...]


_tc_head = pl.pallas_call(
    _tc_head_body,
    out_shape=jax.ShapeDtypeStruct((B, 1), jnp.float32),
)


def kernel(x, edge_index, edge_attr, batch,
           W_node, b_node, W_edge, b_edge,
           W_g1, b_g1, W_g2, b_g2, W_g3, b_g3,
           W_c1, b_c1, W_c2, b_c2):
    src = edge_index[0]
    dst = edge_index[1]
    padi = jnp.full((EPAD - E,), N, jnp.int32)
    srcp = jnp.concatenate([src, padi]).reshape(NW * KPW, CH)
    dstp = jnp.concatenate([dst, padi]).reshape(NW * KPW, CH)
    eap = jnp.concatenate(
        [edge_attr.astype(jnp.bfloat16).astype(jnp.float32),
         jnp.zeros((EPAD - E, DE), jnp.float32)], axis=0)

    onesd = jnp.concatenate(
        [jnp.ones((CH, 8), jnp.float32), jnp.zeros((CH, 8), jnp.float32)],
        axis=1)
    oness = jnp.concatenate(
        [jnp.zeros((CH, 8), jnp.float32), jnp.ones((CH, 8), jnp.float32)],
        axis=1)
    z16 = jnp.zeros((NHP, DE), jnp.float32)
    z64 = jnp.zeros((NSP, H), jnp.float32)
    seld = jnp.zeros((DE, 1), jnp.float32).at[0, 0].set(1.0)
    sels = jnp.zeros((DE, 1), jnp.float32).at[8, 0].set(1.0)

    attr2, degcnt2 = _sc_edge_stats(srcp, dstp, eap, onesd, oness, z16)
    dinv, cnt = _tc_dinv(degcnt2, seld, sels)
    mp1 = _tc_embed(x, attr2, dinv, cnt, W_node, b_node[None, :],
                    W_edge, b_edge[None, :], W_g1)

    acc1 = _sc_aggregate(srcp, dstp, mp1, z64)
    mp2 = _tc_mid(acc1, mp1, dinv, b_g1[None, :], W_g2)
    acc2 = _sc_aggregate(srcp, dstp, mp2, z64)
    mp3 = _tc_mid(acc2, mp2, dinv, b_g2[None, :], W_g3)
    acc3 = _sc_aggregate(srcp, dstp, mp3, z64)

    out = _tc_head(acc3, mp3, dinv, b_g3[None, :], batch[None, :],
                   W_c1, b_c1[None, :], W_c2, b_c2[None, :])
    return out[:, 0]
